# baseline (speedup 1.0000x reference)
import functools

import jax
import jax.numpy as jnp
from jax import lax
from jax.experimental import pallas as pl
from jax.experimental.pallas import tpu as pltpu
from jax.experimental.pallas import tpu_sc as plsc

NC = 2
NS = 16
NW = NC * NS

EPS = 1e-3
ROWS_PER_STREAM = 128
NSLICE = 2


def _sc_gather(table, idx3d, n_rows):
    H = table.shape[1]
    CH = ROWS_PER_STREAM
    chunks_per_w = idx3d.shape[1]
    half_steps = (chunks_per_w - 1) // 2
    mesh = plsc.VectorSubcoreMesh(core_axis_name="c", subcore_axis_name="s")
    mask_hi = jnp.int32(-65536)

    @functools.partial(
        pl.kernel,
        out_type=jax.ShapeDtypeStruct((n_rows // 2, H), jnp.int32),
        mesh=mesh,
        scratch_types=[
            pltpu.VMEM((chunks_per_w, CH), jnp.int32),
            pltpu.VMEM((CH, H), jnp.float32),
            pltpu.VMEM((CH, H), jnp.float32),
            pltpu.VMEM((CH // 2, H), jnp.int32),
            pltpu.VMEM((CH // 2, H), jnp.int32),
            pltpu.SemaphoreType.DMA,
            pltpu.SemaphoreType.DMA,
            pltpu.SemaphoreType.DMA,
            pltpu.SemaphoreType.DMA,
        ],
    )
    def k(table_hbm, idx_hbm, out_hbm, idx_v, rows0, rows1, pk0, pk1,
          g0, g1, s0, s1):
        wid = lax.axis_index("s") * NC + lax.axis_index("c")
        base = wid * chunks_per_w * (CH // 2)
        pltpu.sync_copy(idx_hbm.at[wid], idx_v)

        def fire_gather(c, buf, sem):
            return pltpu.async_copy(table_hbm.at[idx_v.at[c]], buf, sem)

        def wait_gather(c, buf, sem):
            pltpu.make_async_copy(
                table_hbm.at[idx_v.at[c]], buf, sem).wait()

        def fire_store(c, buf, sem):
            return pltpu.async_copy(
                buf, out_hbm.at[pl.ds(base + c * (CH // 2), CH // 2)], sem)

        def wait_store(c, buf, sem):
            pltpu.make_async_copy(
                buf, out_hbm.at[pl.ds(base + c * (CH // 2), CH // 2)],
                sem).wait()

        def convert(rows, pk):
            def body(tp, carry):
                for half in range(2):
                    t = 2 * tp + half
                    for p in range(H // 32):
                        a = lax.bitcast_convert_type(
                            rows[t, pl.ds(32 * p, 16)], jnp.int32)
                        b = lax.bitcast_convert_type(
                            rows[t, pl.ds(32 * p + 16, 16)], jnp.int32)
                        w = lax.shift_right_logical(a, 16) | (b & mask_hi)
                        pk[tp, pl.ds(64 * half + 16 * p, 16)] = w
                return carry
            lax.fori_loop(0, CH // 2, body, 0)

        fire_gather(0, rows0, g0)

        def step(kk, carry):
            c0 = 2 * kk
            c1 = c0 + 1
            fire_gather(c1, rows1, g1)
            wait_gather(c0, rows0, g0)

            @pl.when(kk > 0)
            def _():
                wait_store(c0 - 2, pk0, s0)
            convert(rows0, pk0)
            fire_store(c0, pk0, s0)
            fire_gather(c0 + 2, rows0, g0)
            wait_gather(c1, rows1, g1)

            @pl.when(kk > 0)
            def _():
                wait_store(c1 - 2, pk1, s1)
            convert(rows1, pk1)
            fire_store(c1, pk1, s1)
            return carry

        lax.fori_loop(0, half_steps, step, 0)
        last = chunks_per_w - 1
        wait_gather(last, rows0, g0)
        wait_store(last - 2, pk0, s0)
        convert(rows0, pk0)
        fire_store(last, pk0, s0)
        wait_store(last - 1, pk1, s1)
        wait_store(last, pk0, s0)

    return k(table, idx3d)


def _tc_body_first(g_ref, tt_ref, type_ref, pos_ref, gamma_ref, beta_ref,
                   o_ref):
    BBl, S2, H = g_ref.shape
    w5 = g_ref[...].reshape(BBl, S2, 2, H // 32, 16)
    lo = lax.bitcast_convert_type(w5 << 16, jnp.float32)
    hi = lax.bitcast_convert_type(w5 & jnp.int32(-65536), jnp.float32)
    x = jnp.stack([lo, hi], axis=4).reshape(BBl, 2 * S2, H)
    tt = tt_ref[...].astype(jnp.float32)[..., None]
    t0 = type_ref[0]
    t1 = type_ref[1]
    x = x + t0 + tt * (t1 - t0) + pos_ref[...][None]
    mean = jnp.mean(x, axis=-1, keepdims=True)
    xc = x - mean
    var = jnp.mean(xc * xc, axis=-1, keepdims=True)
    y = xc * lax.rsqrt(var + EPS)
    o_ref[...] = y * gamma_ref[...] + beta_ref[...]


def _tc_body_acc(acc_ref, g_ref, tt_ref, type_ref, pos_ref, gamma_ref,
                 beta_ref, o_ref):
    del acc_ref
    _tc_body_first(g_ref, tt_ref, type_ref, pos_ref, gamma_ref, beta_ref,
                   o_ref)


BB = 8


def _tc_add_ln(acc, off_blocks, gathered, token_type_ids, type_emb,
               pos_slice, gamma, beta, full_b):
    bs, S = token_type_ids.shape
    H = type_emb.shape[1]
    grid = (bs // BB,)
    data_specs = [
        pl.BlockSpec((BB, S // 2, H), lambda i: (i, 0, 0)),
        pl.BlockSpec((BB, S), lambda i: (i, 0)),
        pl.BlockSpec((2, H), lambda i: (0, 0)),
        pl.BlockSpec((S, H), lambda i: (0, 0)),
        pl.BlockSpec((1, H), lambda i: (0, 0)),
        pl.BlockSpec((1, H), lambda i: (0, 0)),
    ]
    out_spec = pl.BlockSpec((BB, S, H), lambda i: (off_blocks + i, 0, 0))
    out_shape = jax.ShapeDtypeStruct((full_b, S, H), jnp.float32)
    params = pltpu.CompilerParams(dimension_semantics=("arbitrary",))
    if acc is None:
        return pl.pallas_call(
            _tc_body_first, grid=grid, in_specs=data_specs,
            out_specs=out_spec, out_shape=out_shape,
            compiler_params=params,
        )(gathered, token_type_ids, type_emb, pos_slice, gamma, beta)
    return pl.pallas_call(
        _tc_body_acc, grid=grid,
        in_specs=[pl.BlockSpec(memory_space=pl.ANY)] + data_specs,
        out_specs=out_spec, out_shape=out_shape,
        input_output_aliases={0: 0},
        compiler_params=params,
    )(acc, gathered, token_type_ids, type_emb, pos_slice, gamma, beta)


def kernel(input_ids, token_type_ids, word_emb, type_emb, pos_emb, gamma, beta):
    B, S = input_ids.shape
    H = word_emb.shape[1]
    bs = B // NSLICE
    n_rows = bs * S
    acc = None
    for i in range(NSLICE):
        ids_i = lax.slice_in_dim(input_ids, i * bs, (i + 1) * bs)
        tt_i = lax.slice_in_dim(token_type_ids, i * bs, (i + 1) * bs)
        idx3d = ids_i.reshape(NW, n_rows // (NW * ROWS_PER_STREAM),
                              ROWS_PER_STREAM)
        g_i = _sc_gather(word_emb, idx3d, n_rows).reshape(bs, S // 2, H)
        acc = _tc_add_ln(acc, i * (bs // BB), g_i, tt_i, type_emb,
                         pos_emb[:S], gamma.reshape(1, H),
                         beta.reshape(1, H), B)
    return acc


# speedup vs baseline: 5.9651x; 5.9651x over previous
import functools

import jax
import jax.numpy as jnp
from jax import lax
from jax.experimental import pallas as pl
from jax.experimental.pallas import tpu as pltpu
from jax.experimental.pallas import tpu_sc as plsc

NC = 2
NS = 16
NW = NC * NS

EPS = 1e-3
ROWS_PER_STREAM = 128
NSLICE = 2


def _sc_gather(table, idx3d, n_rows):
    H = table.shape[1]
    chunks_per_w = idx3d.shape[1]
    mesh = plsc.VectorSubcoreMesh(core_axis_name="c", subcore_axis_name="s")

    @functools.partial(
        pl.kernel,
        out_type=jax.ShapeDtypeStruct((n_rows, H), jnp.float32),
        mesh=mesh,
        scratch_types=[
            pltpu.VMEM((chunks_per_w, ROWS_PER_STREAM), jnp.int32),
            pltpu.VMEM((ROWS_PER_STREAM, H), jnp.float32),
            pltpu.SemaphoreType.DMA,
        ],
    )
    def k(table_hbm, idx_hbm, out_hbm, idx_v, rows_v, sem):
        wid = lax.axis_index("s") * NC + lax.axis_index("c")
        base = wid * chunks_per_w
        pltpu.sync_copy(idx_hbm.at[wid], idx_v)

        def body(i, carry):
            pltpu.async_copy(table_hbm.at[idx_v.at[i]], rows_v, sem).wait()
            row0 = (base + i) * ROWS_PER_STREAM
            pltpu.sync_copy(rows_v, out_hbm.at[pl.ds(row0, ROWS_PER_STREAM)])
            return carry

        lax.fori_loop(0, chunks_per_w, body, 0)

    return k(table, idx3d)


def _tc_body_first(g_ref, tt_ref, type_ref, pos_ref, gamma_ref, beta_ref,
                   o_ref):
    x = g_ref[...]
    tt = tt_ref[...].astype(jnp.float32)[..., None]
    t0 = type_ref[0]
    t1 = type_ref[1]
    x = x + t0 + tt * (t1 - t0) + pos_ref[...][None]
    mean = jnp.mean(x, axis=-1, keepdims=True)
    xc = x - mean
    var = jnp.mean(xc * xc, axis=-1, keepdims=True)
    y = xc * lax.rsqrt(var + EPS)
    o_ref[...] = y * gamma_ref[...] + beta_ref[...]


def _tc_body_acc(acc_ref, g_ref, tt_ref, type_ref, pos_ref, gamma_ref,
                 beta_ref, o_ref):
    del acc_ref
    _tc_body_first(g_ref, tt_ref, type_ref, pos_ref, gamma_ref, beta_ref,
                   o_ref)


BB = 32


def _tc_add_ln(acc, off_blocks, gathered, token_type_ids, type_emb,
               pos_slice, gamma, beta, full_b):
    bs, S = token_type_ids.shape
    H = type_emb.shape[1]
    grid = (bs // BB,)
    data_specs = [
        pl.BlockSpec((BB, S, H), lambda i: (i, 0, 0)),
        pl.BlockSpec((BB, S), lambda i: (i, 0)),
        pl.BlockSpec((2, H), lambda i: (0, 0)),
        pl.BlockSpec((S, H), lambda i: (0, 0)),
        pl.BlockSpec((1, H), lambda i: (0, 0)),
        pl.BlockSpec((1, H), lambda i: (0, 0)),
    ]
    out_spec = pl.BlockSpec((BB, S, H), lambda i: (off_blocks + i, 0, 0))
    out_shape = jax.ShapeDtypeStruct((full_b, S, H), jnp.float32)
    params = pltpu.CompilerParams(dimension_semantics=("arbitrary",))
    if acc is None:
        return pl.pallas_call(
            _tc_body_first, grid=grid, in_specs=data_specs,
            out_specs=out_spec, out_shape=out_shape,
            compiler_params=params,
        )(gathered, token_type_ids, type_emb, pos_slice, gamma, beta)
    return pl.pallas_call(
        _tc_body_acc, grid=grid,
        in_specs=[pl.BlockSpec(memory_space=pl.ANY)] + data_specs,
        out_specs=out_spec, out_shape=out_shape,
        input_output_aliases={0: 0},
        compiler_params=params,
    )(acc, gathered, token_type_ids, type_emb, pos_slice, gamma, beta)


def kernel(input_ids, token_type_ids, word_emb, type_emb, pos_emb, gamma, beta):
    B, S = input_ids.shape
    H = word_emb.shape[1]
    bs = B // NSLICE
    n_rows = bs * S
    acc = None
    for i in range(NSLICE):
        ids_i = lax.slice_in_dim(input_ids, i * bs, (i + 1) * bs)
        tt_i = lax.slice_in_dim(token_type_ids, i * bs, (i + 1) * bs)
        idx3d = ids_i.reshape(NW, n_rows // (NW * ROWS_PER_STREAM),
                              ROWS_PER_STREAM)
        g_i = _sc_gather(word_emb, idx3d, n_rows).reshape(bs, S, H)
        acc = _tc_add_ln(acc, i * (bs // BB), g_i, tt_i, type_emb,
                         pos_emb[:S], gamma.reshape(1, H),
                         beta.reshape(1, H), B)
    return acc


# speedup vs baseline: 6.0568x; 1.0154x over previous
import functools

import jax
import jax.numpy as jnp
from jax import lax
from jax.experimental import pallas as pl
from jax.experimental.pallas import tpu as pltpu
from jax.experimental.pallas import tpu_sc as plsc

NC = 2
NS = 16
NW = NC * NS

EPS = 1e-3
ROWS_PER_STREAM = 128
NSLICE = 2


def _sc_gather(table, idx3d, n_rows):
    H = table.shape[1]
    chunks_per_w = idx3d.shape[1]
    mesh = plsc.VectorSubcoreMesh(core_axis_name="c", subcore_axis_name="s")

    @functools.partial(
        pl.kernel,
        out_type=jax.ShapeDtypeStruct((n_rows, H), jnp.float32),
        mesh=mesh,
        scratch_types=[
            pltpu.VMEM((chunks_per_w, ROWS_PER_STREAM), jnp.int32),
            pltpu.VMEM((ROWS_PER_STREAM, H), jnp.float32),
            pltpu.SemaphoreType.DMA,
        ],
    )
    def k(table_hbm, idx_hbm, out_hbm, idx_v, rows_v, sem):
        wid = lax.axis_index("s") * NC + lax.axis_index("c")
        base = wid * chunks_per_w
        pltpu.sync_copy(idx_hbm.at[wid], idx_v)

        def body(i, carry):
            pltpu.async_copy(table_hbm.at[idx_v.at[i]], rows_v, sem).wait()
            row0 = (base + i) * ROWS_PER_STREAM
            pltpu.sync_copy(rows_v, out_hbm.at[pl.ds(row0, ROWS_PER_STREAM)])
            return carry

        lax.fori_loop(0, chunks_per_w, body, 0)

    return k(table, idx3d)


def _tc_body_first(g_ref, tt_ref, type_ref, pos_ref, gamma_ref, beta_ref,
                   o_ref):
    x = g_ref[...]
    tt = tt_ref[...].astype(jnp.float32)[..., None]
    t0 = type_ref[0]
    t1 = type_ref[1]
    x = x + t0 + tt * (t1 - t0) + pos_ref[...][None]
    mean = jnp.mean(x, axis=-1, keepdims=True)
    xc = x - mean
    var = jnp.mean(xc * xc, axis=-1, keepdims=True)
    y = xc * lax.rsqrt(var + EPS)
    o_ref[...] = y * gamma_ref[...] + beta_ref[...]


def _tc_body_acc(acc_ref, g_ref, tt_ref, type_ref, pos_ref, gamma_ref,
                 beta_ref, o_ref):
    del acc_ref
    _tc_body_first(g_ref, tt_ref, type_ref, pos_ref, gamma_ref, beta_ref,
                   o_ref)


BB = 64


def _tc_add_ln(acc, off_blocks, gathered, token_type_ids, type_emb,
               pos_slice, gamma, beta, full_b):
    bs, S = token_type_ids.shape
    H = type_emb.shape[1]
    grid = (bs // BB,)
    data_specs = [
        pl.BlockSpec((BB, S, H), lambda i: (i, 0, 0)),
        pl.BlockSpec((BB, S), lambda i: (i, 0)),
        pl.BlockSpec((2, H), lambda i: (0, 0)),
        pl.BlockSpec((S, H), lambda i: (0, 0)),
        pl.BlockSpec((1, H), lambda i: (0, 0)),
        pl.BlockSpec((1, H), lambda i: (0, 0)),
    ]
    out_spec = pl.BlockSpec((BB, S, H), lambda i: (off_blocks + i, 0, 0))
    out_shape = jax.ShapeDtypeStruct((full_b, S, H), jnp.float32)
    params = pltpu.CompilerParams(dimension_semantics=("arbitrary",))
    if acc is None:
        return pl.pallas_call(
            _tc_body_first, grid=grid, in_specs=data_specs,
            out_specs=out_spec, out_shape=out_shape,
            compiler_params=params,
        )(gathered, token_type_ids, type_emb, pos_slice, gamma, beta)
    return pl.pallas_call(
        _tc_body_acc, grid=grid,
        in_specs=[pl.BlockSpec(memory_space=pl.ANY)] + data_specs,
        out_specs=out_spec, out_shape=out_shape,
        input_output_aliases={0: 0},
        compiler_params=params,
    )(acc, gathered, token_type_ids, type_emb, pos_slice, gamma, beta)


def kernel(input_ids, token_type_ids, word_emb, type_emb, pos_emb, gamma, beta):
    B, S = input_ids.shape
    H = word_emb.shape[1]
    bs = B // NSLICE
    n_rows = bs * S
    acc = None
    for i in range(NSLICE):
        ids_i = lax.slice_in_dim(input_ids, i * bs, (i + 1) * bs)
        tt_i = lax.slice_in_dim(token_type_ids, i * bs, (i + 1) * bs)
        idx3d = ids_i.reshape(NW, n_rows // (NW * ROWS_PER_STREAM),
                              ROWS_PER_STREAM)
        g_i = _sc_gather(word_emb, idx3d, n_rows).reshape(bs, S, H)
        acc = _tc_add_ln(acc, i * (bs // BB), g_i, tt_i, type_emb,
                         pos_emb[:S], gamma.reshape(1, H),
                         beta.reshape(1, H), B)
    return acc


# speedup vs baseline: 6.0636x; 1.0011x over previous
import functools

import jax
import jax.numpy as jnp
from jax import lax
from jax.experimental import pallas as pl
from jax.experimental.pallas import tpu as pltpu
from jax.experimental.pallas import tpu_sc as plsc

NC = 2
NS = 16
NW = NC * NS

EPS = 1e-3
ROWS_PER_STREAM = 128
NSLICE = 2


def _sc_gather(table, idx3d, n_rows):
    H = table.shape[1]
    chunks_per_w = idx3d.shape[1]
    mesh = plsc.VectorSubcoreMesh(core_axis_name="c", subcore_axis_name="s")

    @functools.partial(
        pl.kernel,
        out_type=jax.ShapeDtypeStruct((n_rows, H), jnp.float32),
        mesh=mesh,
        scratch_types=[
            pltpu.VMEM((chunks_per_w, ROWS_PER_STREAM), jnp.int32),
            pltpu.VMEM((ROWS_PER_STREAM, H), jnp.float32),
            pltpu.SemaphoreType.DMA,
        ],
    )
    def k(table_hbm, idx_hbm, out_hbm, idx_v, rows_v, sem):
        wid = lax.axis_index("s") * NC + lax.axis_index("c")
        base = wid * chunks_per_w
        pltpu.sync_copy(idx_hbm.at[wid], idx_v)

        def body(i, carry):
            pltpu.async_copy(table_hbm.at[idx_v.at[i]], rows_v, sem).wait()
            row0 = (base + i) * ROWS_PER_STREAM
            pltpu.sync_copy(rows_v, out_hbm.at[pl.ds(row0, ROWS_PER_STREAM)])
            return carry

        lax.fori_loop(0, chunks_per_w, body, 0)

    return k(table, idx3d)


def _tc_body_first(g_ref, tt_ref, type_ref, pos_ref, gamma_ref, beta_ref,
                   o_ref):
    x = g_ref[...]
    tt = tt_ref[...].astype(jnp.float32)[..., None]
    t0 = type_ref[0]
    t1 = type_ref[1]
    x = x + t0 + tt * (t1 - t0) + pos_ref[...][None]
    mean = jnp.mean(x, axis=-1, keepdims=True)
    xc = x - mean
    var = jnp.mean(xc * xc, axis=-1, keepdims=True)
    y = xc * lax.rsqrt(var + EPS)
    o_ref[...] = y * gamma_ref[...] + beta_ref[...]


def _tc_body_acc(acc_ref, g_ref, tt_ref, type_ref, pos_ref, gamma_ref,
                 beta_ref, o_ref):
    del acc_ref
    _tc_body_first(g_ref, tt_ref, type_ref, pos_ref, gamma_ref, beta_ref,
                   o_ref)


BB = 64


def _tc_add_ln(acc, off_blocks, gathered, token_type_ids, type_emb,
               pos_slice, gamma, beta, full_b):
    bs, S = token_type_ids.shape
    H = type_emb.shape[1]
    grid = (bs // BB,)
    data_specs = [
        pl.BlockSpec((BB, S, H), lambda i: (i, 0, 0)),
        pl.BlockSpec((BB, S), lambda i: (i, 0)),
        pl.BlockSpec((2, H), lambda i: (0, 0)),
        pl.BlockSpec((S, H), lambda i: (0, 0)),
        pl.BlockSpec((1, H), lambda i: (0, 0)),
        pl.BlockSpec((1, H), lambda i: (0, 0)),
    ]
    out_spec = pl.BlockSpec((BB, S, H), lambda i: (off_blocks + i, 0, 0))
    out_shape = jax.ShapeDtypeStruct((full_b, S, H), jnp.float32)
    params = pltpu.CompilerParams(dimension_semantics=("parallel",))
    if acc is None:
        return pl.pallas_call(
            _tc_body_first, grid=grid, in_specs=data_specs,
            out_specs=out_spec, out_shape=out_shape,
            compiler_params=params,
        )(gathered, token_type_ids, type_emb, pos_slice, gamma, beta)
    return pl.pallas_call(
        _tc_body_acc, grid=grid,
        in_specs=[pl.BlockSpec(memory_space=pl.ANY)] + data_specs,
        out_specs=out_spec, out_shape=out_shape,
        input_output_aliases={0: 0},
        compiler_params=params,
    )(acc, gathered, token_type_ids, type_emb, pos_slice, gamma, beta)


def kernel(input_ids, token_type_ids, word_emb, type_emb, pos_emb, gamma, beta):
    B, S = input_ids.shape
    H = word_emb.shape[1]
    bs = B // NSLICE
    n_rows = bs * S
    acc = None
    for i in range(NSLICE):
        ids_i = lax.slice_in_dim(input_ids, i * bs, (i + 1) * bs)
        tt_i = lax.slice_in_dim(token_type_ids, i * bs, (i + 1) * bs)
        idx3d = ids_i.reshape(NW, n_rows // (NW * ROWS_PER_STREAM),
                              ROWS_PER_STREAM)
        g_i = _sc_gather(word_emb, idx3d, n_rows).reshape(bs, S, H)
        acc = _tc_add_ln(acc, i * (bs // BB), g_i, tt_i, type_emb,
                         pos_emb[:S], gamma.reshape(1, H),
                         beta.reshape(1, H), B)
    return acc
